# baseline (speedup 1.0000x reference)
import functools

import jax
import jax.numpy as jnp
from jax import lax
from jax.experimental import pallas as pl
from jax.experimental.pallas import tpu as pltpu
from jax.experimental.pallas import tpu_sc as plsc

UNITS = 128
VOCAB = 1000
ROWS = VOCAB + 1
DPAD = 1024
BATCH = 4096
SEQ = 20
TOK = BATCH * SEQ

NC, NS, L = 2, 16, 16
NW = NC * NS
BPW = BATCH // NW
STRIDE = 32
XPW = BPW * STRIDE
GR = 24
NWIN = ROWS // L
TAIL = ROWS - L


def _gram_body(w_ref, wp_ref, g_ref):
    g_ref[...] = lax.dot_general(
        w_ref[...], wp_ref[...], (((1,), (1,)), ((), ())),
        preferred_element_type=jnp.float32,
    )


def _gram(w):
    wp = jnp.pad(w, ((0, DPAD - ROWS), (0, 0)))
    return pl.pallas_call(
        _gram_body,
        out_shape=jax.ShapeDtypeStruct((ROWS, DPAD), jnp.float32),
    )(w, wp)


def _gather_body(x_hbm, g_hbm, out_hbm,
                 xbuf, idxbuf, rows0, rows1, ob0, ob1,
                 gsem0, gsem1, wsem0, wsem1):
    wid = lax.axis_index("s") * NC + lax.axis_index("c")
    b0 = wid * BPW

    pltpu.sync_copy(x_hbm.at[pl.ds(wid * XPW, XPW)], xbuf)

    def conv(j, c):
        sl = pl.ds(j * L, L)
        idxbuf[sl] = (xbuf[sl] * float(VOCAB)).astype(jnp.int32)
        return c

    lax.fori_loop(0, XPW // L, conv, 0)

    rows = (rows0, rows1)
    obufs = (ob0, ob1)
    gsems = (gsem0, gsem1)
    wsems = (wsem0, wsem1)

    def gdesc(b, s):
        idxsl = idxbuf.at[pl.ds(b * STRIDE, GR)]
        return g_hbm.at[idxsl], rows[s], gsems[s]

    def issue(b, s):
        src_d, dst_d, sem_d = gdesc(b, s)
        pltpu.async_copy(src_d, dst_d, sem_d)

    def process(b, s):
        src_d, dst_d, sem_d = gdesc(b, s)
        pltpu.make_async_copy(src_d, dst_d, sem_d).wait()

        @plsc.parallel_loop(0, SEQ, 1, unroll=2)
        def _(r):
            src = rows[s].at[r]
            ob = obufs[s]
            for j in range(NWIN):
                sl = pl.ds(j * L, L)
                ob[r, sl] = src[sl]
            tl = pl.ds(TAIL, L)
            ob[r, tl] = src[tl]

        pltpu.sync_copy(obufs[s], out_hbm.at[b0 + b])

    issue(0, 0)

    def pair(i, carry):
        g0 = i * 2
        issue(g0 + 1, 1)
        process(g0, 0)

        @pl.when(i + 1 < BPW // 2)
        def _():
            issue(g0 + 2, 0)

        process(g0 + 1, 1)
        return carry

    lax.fori_loop(0, BPW // 2, pair, 0)


_gather = functools.partial(
    pl.kernel,
    out_type=jax.ShapeDtypeStruct((BATCH, SEQ, ROWS), jnp.float32),
    mesh=plsc.VectorSubcoreMesh(core_axis_name="c", subcore_axis_name="s"),
    scratch_types=[
        pltpu.VMEM((XPW,), jnp.float32),
        pltpu.VMEM((XPW,), jnp.int32),
        pltpu.VMEM((GR, DPAD), jnp.float32),
        pltpu.VMEM((GR, DPAD), jnp.float32),
        pltpu.VMEM((SEQ, ROWS), jnp.float32),
        pltpu.VMEM((SEQ, ROWS), jnp.float32),
        pltpu.SemaphoreType.DMA,
        pltpu.SemaphoreType.DMA,
        pltpu.SemaphoreType.DMA,
        pltpu.SemaphoreType.DMA,
    ],
)(_gather_body)


def kernel(x, c, we_item):
    del c
    g = _gram(we_item)
    xs = jnp.pad(x.reshape(BATCH, SEQ), ((0, 0), (0, STRIDE - SEQ)))
    return _gather(xs.reshape(BATCH * STRIDE), g)


# speedup vs baseline: 2.5342x; 2.5342x over previous
import functools

import jax
import jax.numpy as jnp
from jax import lax
from jax.experimental import pallas as pl
from jax.experimental.pallas import tpu as pltpu
from jax.experimental.pallas import tpu_sc as plsc

UNITS = 128
VOCAB = 1000
ROWS = VOCAB + 1
DPAD = 1024
BATCH = 4096
SEQ = 20
TOK = BATCH * SEQ

NC, NS, L = 2, 16, 16
NW = NC * NS
BPW = BATCH // NW
STRIDE = 24
XPW = BPW * STRIDE
GR = 24
NWIN = ROWS // L
TAIL = ROWS - L


def _gram_body(w_ref, wp_ref, g_ref):
    g_ref[...] = lax.dot_general(
        w_ref[...], wp_ref[...], (((1,), (1,)), ((), ())),
        preferred_element_type=jnp.float32,
    )


def _gram(w):
    wp = jnp.pad(w, ((0, DPAD - ROWS), (0, 0)))
    return pl.pallas_call(
        _gram_body,
        out_shape=jax.ShapeDtypeStruct((ROWS, DPAD), jnp.float32),
    )(w, wp)


def _gather_body(x_hbm, g_hbm, out_hbm,
                 xbuf, idxbuf, rows0, rows1, ob0, ob1,
                 gsem0, gsem1, wsem0, wsem1):
    wid = lax.axis_index("s") * NC + lax.axis_index("c")
    b0 = wid * BPW

    pltpu.sync_copy(x_hbm.at[pl.ds(wid * XPW, XPW)], xbuf)

    def conv(j, c):
        sl = pl.ds(j * L, L)
        idxbuf[sl] = (xbuf[sl] * float(VOCAB)).astype(jnp.int32)
        return c

    lax.fori_loop(0, XPW // L, conv, 0)

    rows = (rows0, rows1)
    obufs = (ob0, ob1)
    gsems = (gsem0, gsem1)
    wsems = (wsem0, wsem1)

    def gdesc(b, s):
        idxsl = idxbuf.at[pl.ds(b * STRIDE, GR)]
        return g_hbm.at[idxsl], rows[s], gsems[s]

    def issue(b, s):
        src_d, dst_d, sem_d = gdesc(b, s)
        pltpu.async_copy(src_d, dst_d, sem_d)

    def process(b, s):
        src_d, dst_d, sem_d = gdesc(b, s)
        pltpu.make_async_copy(src_d, dst_d, sem_d).wait()

        @plsc.parallel_loop(0, SEQ, 1, unroll=2)
        def _(r):
            src = rows[s].at[r]
            ob = obufs[s]
            for j in range(NWIN):
                sl = pl.ds(j * L, L)
                ob[r, sl] = src[sl]
            tl = pl.ds(TAIL, L)
            ob[r, tl] = src[tl]

        pltpu.sync_copy(obufs[s], out_hbm.at[b0 + b])

    issue(0, 0)

    def pair(i, carry):
        g0 = i * 2
        issue(g0 + 1, 1)
        process(g0, 0)

        @pl.when(i + 1 < BPW // 2)
        def _():
            issue(g0 + 2, 0)

        process(g0 + 1, 1)
        return carry

    lax.fori_loop(0, BPW // 2, pair, 0)


_gather = functools.partial(
    pl.kernel,
    out_type=jax.ShapeDtypeStruct((BATCH, SEQ, ROWS), jnp.float32),
    mesh=plsc.VectorSubcoreMesh(core_axis_name="c", subcore_axis_name="s"),
    scratch_types=[
        pltpu.VMEM((XPW,), jnp.float32),
        pltpu.VMEM((XPW,), jnp.int32),
        pltpu.VMEM((GR, DPAD), jnp.float32),
        pltpu.VMEM((GR, DPAD), jnp.float32),
        pltpu.VMEM((SEQ, ROWS), jnp.float32),
        pltpu.VMEM((SEQ, ROWS), jnp.float32),
        pltpu.SemaphoreType.DMA,
        pltpu.SemaphoreType.DMA,
        pltpu.SemaphoreType.DMA,
        pltpu.SemaphoreType.DMA,
    ],
)(_gather_body)


def kernel(x, c, we_item):
    del c
    g = _gram(we_item)
    x2 = x.reshape(BATCH, SEQ)
    xs = jnp.concatenate([x2, x2[:, : STRIDE - SEQ]], axis=1)
    return _gather(xs.reshape(BATCH * STRIDE), g)


# speedup vs baseline: 2.5653x; 1.0123x over previous
import functools

import jax
import jax.numpy as jnp
from jax import lax
from jax.experimental import pallas as pl
from jax.experimental.pallas import tpu as pltpu
from jax.experimental.pallas import tpu_sc as plsc

UNITS = 128
VOCAB = 1000
ROWS = VOCAB + 1
DPAD = 1024
BATCH = 4096
SEQ = 20
TOK = BATCH * SEQ

NC, NS, L = 2, 16, 16
NW = NC * NS
BPW = BATCH // NW
STRIDE = 24
XPW = BPW * STRIDE
GR = 24
NWIN = ROWS // L
TAIL = ROWS - L


def _gram_body(w_ref, wp_ref, g_ref):
    g_ref[...] = lax.dot_general(
        w_ref[...], wp_ref[...], (((1,), (1,)), ((), ())),
        preferred_element_type=jnp.float32,
    )


def _gram(w):
    wp = jnp.pad(w, ((0, DPAD - ROWS), (0, 0)))
    return pl.pallas_call(
        _gram_body,
        out_shape=jax.ShapeDtypeStruct((ROWS, DPAD), jnp.float32),
    )(w, wp)


def _gather_body(x_hbm, g_hbm, out_hbm,
                 xbuf, idxbuf, rows0, rows1, ob0, ob1,
                 gsem0, gsem1, wsem0, wsem1):
    wid = lax.axis_index("s") * NC + lax.axis_index("c")
    b0 = wid * BPW

    pltpu.sync_copy(x_hbm.at[pl.ds(wid * XPW, XPW)], xbuf)

    def conv(j, c):
        sl = pl.ds(j * L, L)
        idxbuf[sl] = (xbuf[sl] * float(VOCAB)).astype(jnp.int32)
        return c

    lax.fori_loop(0, XPW // L, conv, 0)

    rows = (rows0, rows1)
    obufs = (ob0, ob1)
    gsems = (gsem0, gsem1)
    wsems = (wsem0, wsem1)

    def gdesc(b, s):
        idxsl = idxbuf.at[pl.ds(b * STRIDE, GR)]
        return g_hbm.at[idxsl], rows[s], gsems[s]

    def issue(b, s):
        src_d, dst_d, sem_d = gdesc(b, s)
        pltpu.async_copy(src_d, dst_d, sem_d)

    def process(b, s):
        src_d, dst_d, sem_d = gdesc(b, s)
        pltpu.make_async_copy(src_d, dst_d, sem_d).wait()

        @pl.when(b >= 2)
        def _():
            pltpu.make_async_copy(
                obufs[s], out_hbm.at[b0 + b - 2], wsems[s]).wait()

        @plsc.parallel_loop(0, SEQ, 1, unroll=2)
        def _(r):
            src = rows[s].at[r]
            ob = obufs[s]
            for j in range(NWIN):
                sl = pl.ds(j * L, L)
                ob[r, sl] = src[sl]
            tl = pl.ds(TAIL, L)
            ob[r, tl] = src[tl]

        pltpu.async_copy(obufs[s], out_hbm.at[b0 + b], wsems[s])

    issue(0, 0)

    def pair(i, carry):
        g0 = i * 2
        issue(g0 + 1, 1)
        process(g0, 0)

        @pl.when(i + 1 < BPW // 2)
        def _():
            issue(g0 + 2, 0)

        process(g0 + 1, 1)
        return carry

    lax.fori_loop(0, BPW // 2, pair, 0)

    pltpu.make_async_copy(obufs[0], out_hbm.at[b0 + BPW - 2], wsems[0]).wait()
    pltpu.make_async_copy(obufs[1], out_hbm.at[b0 + BPW - 1], wsems[1]).wait()


_gather = functools.partial(
    pl.kernel,
    out_type=jax.ShapeDtypeStruct((BATCH, SEQ, ROWS), jnp.float32),
    mesh=plsc.VectorSubcoreMesh(core_axis_name="c", subcore_axis_name="s"),
    scratch_types=[
        pltpu.VMEM((XPW,), jnp.float32),
        pltpu.VMEM((XPW,), jnp.int32),
        pltpu.VMEM((GR, DPAD), jnp.float32),
        pltpu.VMEM((GR, DPAD), jnp.float32),
        pltpu.VMEM((SEQ, ROWS), jnp.float32),
        pltpu.VMEM((SEQ, ROWS), jnp.float32),
        pltpu.SemaphoreType.DMA,
        pltpu.SemaphoreType.DMA,
        pltpu.SemaphoreType.DMA,
        pltpu.SemaphoreType.DMA,
    ],
)(_gather_body)


def kernel(x, c, we_item):
    del c
    g = _gram(we_item)
    x2 = x.reshape(BATCH, SEQ)
    xs = jnp.concatenate([x2, x2[:, : STRIDE - SEQ]], axis=1)
    return _gather(xs.reshape(BATCH * STRIDE), g)
